# baseline (device time: 269596 ns/iter reference)
import jax
import jax.numpy as jnp
from jax import lax
from jax.experimental import pallas as pl
from jax.experimental.pallas import tpu as pltpu

N_SLOTS = 2


def kernel(O, Wo):
    B, S, Hl, D = O.shape
    K = Hl * D
    N = Wo.shape[1]
    s_half = S // 2
    n_chunks = 2
    n_blk = N // n_chunks
    T = n_chunks * B

    A = O.reshape(B, 2, s_half, K).astype(jnp.bfloat16)
    W = Wo.astype(jnp.bfloat16)

    def body(a_ref, w_ref, out_ref,
             res_buf, recv_buf, send_sems, recv_sems, credit_sem):
        t = pl.program_id(0)
        slot = t % N_SLOTS
        prev_slot = (t + 1) % N_SLOTS

        my_x = lax.axis_index("x")
        my_y = lax.axis_index("y")
        my_z = lax.axis_index("z")
        partner = (1 - my_x, my_y, my_z)

        def rdma_for(s):
            return pltpu.make_async_remote_copy(
                src_ref=res_buf.at[s, 1 - my_x],
                dst_ref=recv_buf.at[s],
                send_sem=send_sems.at[s],
                recv_sem=recv_sems.at[s],
                device_id=partner,
                device_id_type=pl.DeviceIdType.MESH,
            )

        @pl.when(t == 0)
        def _():
            barrier = pltpu.get_barrier_semaphore()
            pl.semaphore_signal(
                barrier, inc=1,
                device_id=partner, device_id_type=pl.DeviceIdType.MESH,
            )
            pl.semaphore_wait(barrier, 1)

        @pl.when(jnp.logical_and(t >= N_SLOTS, t < T))
        def _():
            rdma_for(slot).wait_send()
            pl.semaphore_wait(credit_sem, 1)

        @pl.when(t < T)
        def _():
            av = a_ref[0].reshape(2 * s_half, K)
            res = jnp.dot(av, w_ref[...], preferred_element_type=jnp.float32)
            res_buf[slot] = res.astype(jnp.bfloat16).reshape(2, s_half, n_blk)
            rdma_for(slot).start()

        @pl.when(t > 0)
        def _():
            rdma_for(prev_slot).wait_recv()
            out_ref[0] = res_buf[prev_slot, my_x].astype(jnp.float32) + \
                recv_buf[prev_slot].astype(jnp.float32)

        @pl.when(jnp.logical_and(t > 0, t - 1 < T - N_SLOTS))
        def _():
            pl.semaphore_signal(
                credit_sem, inc=1,
                device_id=partner, device_id_type=pl.DeviceIdType.MESH,
            )

        @pl.when(t == T)
        def _():
            for s in range(N_SLOTS):
                rdma_for(s).wait_send()

    def a_map(t):
        te = jnp.minimum(t, T - 1)
        return (te % B, 0, 0, 0)

    def w_map(t):
        te = jnp.minimum(t, T - 1)
        return (0, te // B)

    def out_map(t):
        to = jnp.maximum(t - 1, 0)
        return (to % B, 0, to // B)

    return pl.pallas_call(
        body,
        grid=(T + 1,),
        out_shape=jax.ShapeDtypeStruct((B, s_half, N), jnp.float32),
        in_specs=[
            pl.BlockSpec((1, 2, s_half, K), a_map, memory_space=pltpu.VMEM),
            pl.BlockSpec((K, n_blk), w_map, memory_space=pltpu.VMEM),
        ],
        out_specs=pl.BlockSpec((1, s_half, n_blk), out_map,
                               memory_space=pltpu.VMEM),
        scratch_shapes=[
            pltpu.VMEM((N_SLOTS, 2, s_half, n_blk), jnp.bfloat16),
            pltpu.VMEM((N_SLOTS, s_half, n_blk), jnp.bfloat16),
            pltpu.SemaphoreType.DMA((N_SLOTS,)),
            pltpu.SemaphoreType.DMA((N_SLOTS,)),
            pltpu.SemaphoreType.REGULAR,
        ],
        compiler_params=pltpu.CompilerParams(
            collective_id=0,
            dimension_semantics=("arbitrary",),
            vmem_limit_bytes=60 * 1024 * 1024,
        ),
    )(A, W)


# device time: 254293 ns/iter; 1.0602x vs baseline; 1.0602x over previous
import jax
import jax.numpy as jnp
from jax import lax
from jax.experimental import pallas as pl
from jax.experimental.pallas import tpu as pltpu

N_SLOTS = 2


def kernel(O, Wo):
    B, S, Hl, D = O.shape
    K = Hl * D
    N = Wo.shape[1]
    s_half = S // 2
    n_chunks = 2
    n_blk = N // n_chunks
    T = n_chunks * B

    A = O.reshape(B, 2, s_half, K).astype(jnp.bfloat16)
    W = Wo.astype(jnp.bfloat16)

    def body(a_ref, w_ref, out_ref,
             res_buf, recv_buf, send_sems, recv_sems, credit_sem):
        t = pl.program_id(0)
        slot = t % N_SLOTS
        prev_slot = (t + 1) % N_SLOTS

        my_x = lax.axis_index("x")
        my_y = lax.axis_index("y")
        my_z = lax.axis_index("z")
        partner = (1 - my_x, my_y, my_z)

        def rdma_for(s):
            return pltpu.make_async_remote_copy(
                src_ref=res_buf.at[s, 1 - my_x],
                dst_ref=recv_buf.at[s],
                send_sem=send_sems.at[s],
                recv_sem=recv_sems.at[s],
                device_id=partner,
                device_id_type=pl.DeviceIdType.MESH,
            )

        @pl.when(t == 0)
        def _():
            barrier = pltpu.get_barrier_semaphore()
            pl.semaphore_signal(
                barrier, inc=1,
                device_id=partner, device_id_type=pl.DeviceIdType.MESH,
            )
            pl.semaphore_wait(barrier, 1)

        @pl.when(jnp.logical_and(t >= N_SLOTS, t < T))
        def _():
            rdma_for(slot).wait_send()
            pl.semaphore_wait(credit_sem, 1)

        @pl.when(t < T)
        def _():
            res_buf[slot, 1 - my_x] = jnp.dot(
                a_ref[0, 1 - my_x], w_ref[...],
                preferred_element_type=jnp.float32,
            ).astype(jnp.bfloat16)
            rdma_for(slot).start()
            res_buf[slot, my_x] = jnp.dot(
                a_ref[0, my_x], w_ref[...],
                preferred_element_type=jnp.float32,
            ).astype(jnp.bfloat16)

        @pl.when(t > 0)
        def _():
            rdma_for(prev_slot).wait_recv()
            out_ref[0] = (
                res_buf[prev_slot, my_x].astype(jnp.float32)
                + recv_buf[prev_slot].astype(jnp.float32)
            ).astype(jnp.bfloat16)

        @pl.when(jnp.logical_and(t > 0, t - 1 < T - N_SLOTS))
        def _():
            pl.semaphore_signal(
                credit_sem, inc=1,
                device_id=partner, device_id_type=pl.DeviceIdType.MESH,
            )

        @pl.when(t == T)
        def _():
            for s in range(N_SLOTS):
                rdma_for(s).wait_send()

    def a_map(t):
        te = jnp.minimum(t, T - 1)
        return (te % B, 0, 0, 0)

    def w_map(t):
        te = jnp.minimum(t, T - 1)
        return (0, te // B)

    def out_map(t):
        to = jnp.maximum(t - 1, 0)
        return (to % B, 0, to // B)

    return pl.pallas_call(
        body,
        grid=(T + 1,),
        out_shape=jax.ShapeDtypeStruct((B, s_half, N), jnp.bfloat16),
        in_specs=[
            pl.BlockSpec((1, 2, s_half, K), a_map, memory_space=pltpu.VMEM),
            pl.BlockSpec((K, n_blk), w_map, memory_space=pltpu.VMEM),
        ],
        out_specs=pl.BlockSpec((1, s_half, n_blk), out_map,
                               memory_space=pltpu.VMEM),
        scratch_shapes=[
            pltpu.VMEM((N_SLOTS, 2, s_half, n_blk), jnp.bfloat16),
            pltpu.VMEM((N_SLOTS, s_half, n_blk), jnp.bfloat16),
            pltpu.SemaphoreType.DMA((N_SLOTS,)),
            pltpu.SemaphoreType.DMA((N_SLOTS,)),
            pltpu.SemaphoreType.REGULAR,
        ],
        compiler_params=pltpu.CompilerParams(
            collective_id=0,
            dimension_semantics=("arbitrary",),
            vmem_limit_bytes=60 * 1024 * 1024,
        ),
    )(A, W)
